# baseline (speedup 1.0000x reference)
import functools
import math

import jax
import jax.numpy as jnp
from jax import lax
from jax.experimental import pallas as pl
from jax.experimental.pallas import tpu as pltpu
from jax.experimental.pallas import tpu_sc as plsc

_NSC = 2
_NTILE = 16
_NSUB = _NSC * _NTILE
_CHUNK = 128
_LEAK = 0.01


@functools.lru_cache(maxsize=None)
def _make_segsum(n_nodes, n_pad, k_chunks, hid):
    rows_per_tile = n_pad // _NTILE
    zc = rows_per_tile // _CHUNK
    k_half = k_chunks // 2
    mesh = plsc.VectorSubcoreMesh(core_axis_name="c", subcore_axis_name="s")

    @functools.partial(
        pl.kernel,
        mesh=mesh,
        out_type=jax.ShapeDtypeStruct((_NSC, n_pad, hid), jnp.float32),
        scratch_types=[
            pltpu.VMEM((k_half, _CHUNK), jnp.int32),
            pltpu.VMEM((k_half, _CHUNK), jnp.int32),
            pltpu.VMEM((_CHUNK, hid), jnp.float32),
            pltpu.VMEM((_CHUNK, hid), jnp.float32),
            pltpu.VMEM_SHARED((n_pad, hid), jnp.float32),
            pltpu.SemaphoreType.DMA,
            pltpu.SemaphoreType.DMA,
        ],
    )
    def segsum(hd_hbm, srcp_hbm, dstp_hbm, out_hbm,
               src_v, dst_v, buf0, buf1, acc, sem0, sem1):
        c = lax.axis_index("c")
        s = lax.axis_index("s")
        w = c * _NTILE + s

        def _zrow(r, carry):
            for cc in range(hid // 16):
                buf0[r, pl.ds(cc * 16, 16)] = jnp.zeros((16,), jnp.float32)
            return carry
        lax.fori_loop(0, _CHUNK, _zrow, 0)
        base = s * rows_per_tile
        for k in range(zc):
            pltpu.sync_copy(buf0, acc.at[pl.ds(base + k * _CHUNK, _CHUNK)])
        plsc.subcore_barrier()

        def _group(g, carry):
            j0 = g * 2
            cp0 = pltpu.async_copy(hd_hbm.at[src_v.at[j0]], buf0, sem0)
            cp1 = pltpu.async_copy(hd_hbm.at[src_v.at[j0 + 1]], buf1, sem1)
            cp0.wait()
            pltpu.sync_copy(buf0, acc.at[dst_v.at[j0]], add=True)
            cp1.wait()
            pltpu.sync_copy(buf1, acc.at[dst_v.at[j0 + 1]], add=True)
            return carry

        for half in range(2):
            pltpu.sync_copy(srcp_hbm.at[w, pl.ds(half * k_half, k_half)], src_v)
            pltpu.sync_copy(dstp_hbm.at[w, pl.ds(half * k_half, k_half)], dst_v)
            lax.fori_loop(0, k_half // 2, _group, 0)
        plsc.subcore_barrier()

        for k in range(zc):
            pltpu.sync_copy(acc.at[pl.ds(base + k * _CHUNK, _CHUNK)], buf0)
            pltpu.sync_copy(buf0, out_hbm.at[c, pl.ds(base + k * _CHUNK, _CHUNK)])

    return segsum


def _init_body(nf_ref, wn_ref, bn_ref, h0_ref, cs_ref):
    i = pl.program_id(0)
    h0 = jnp.dot(nf_ref[...], wn_ref[...],
                 preferred_element_type=jnp.float32) + bn_ref[...]
    h0_ref[...] = h0

    @pl.when(i == 0)
    def _():
        cs_ref[...] = jnp.zeros_like(cs_ref)
    cs_ref[...] += jnp.sum(h0, axis=0, keepdims=True)


def _combine_body(p0_ref, p1_ref, deg_ref, h1_ref, hd1_ref):
    h1 = p0_ref[0] + p1_ref[0]
    h1_ref[...] = h1
    hd1_ref[...] = h1 / deg_ref[...]


def _make_layer_body(need_hd):
    def _layer_body(p0_ref, p1_ref, h_ref, deg_ref, w_ref, b_ref, *out_refs):
        i = pl.program_id(0)
        agg = p0_ref[0] + p1_ref[0] + h_ref[...]
        hn = jnp.dot(agg, w_ref[...],
                     preferred_element_type=jnp.float32) + b_ref[...]
        hn = jnp.where(hn > 0, hn, _LEAK * hn)
        if need_hd:
            hn_ref, hdn_ref, cs_ref = out_refs
            hdn_ref[...] = hn / deg_ref[...]
        else:
            hn_ref, cs_ref = out_refs
        hn_ref[...] = hn

        @pl.when(i == 0)
        def _():
            cs_ref[...] = jnp.zeros_like(cs_ref)
        cs_ref[...] += jnp.sum(hn, axis=0, keepdims=True)
    return _layer_body


def _make_head_body(n_nodes):
    def _head_body(c0, c1, c2, c3, wp0, wp1, wp2, wp3, bp, wc, bc, out_ref):
        inv_n = 1.0 / n_nodes
        feat = (jnp.dot(c0[...] * inv_n, wp0[...], preferred_element_type=jnp.float32)
                + jnp.dot(c1[...] * inv_n, wp1[...], preferred_element_type=jnp.float32)
                + jnp.dot(c2[...] * inv_n, wp2[...], preferred_element_type=jnp.float32)
                + jnp.dot(c3[...] * inv_n, wp3[...], preferred_element_type=jnp.float32)
                + bp[...])
        logits = jnp.dot(feat, wc[...], preferred_element_type=jnp.float32) + bc[...]
        m = jnp.max(logits, axis=-1, keepdims=True)
        ex = jnp.exp(logits - m)
        out_ref[...] = ex / jnp.sum(ex, axis=-1, keepdims=True)
    return _head_body


def kernel(node_feat, edge_feat, edge_index, degree, base_data,
           Wn, bn, We, be, Wgcn, bgcn, Wpred, bpred, Wcls, bcls):
    n, d_node = node_feat.shape
    hid = Wn.shape[1]
    e = edge_index.shape[1]
    cls = Wcls.shape[1]
    layers = Wpred.shape[0] // hid - 1

    per_sub_chunks = math.ceil(e / (_NSUB * _CHUNK))
    per_sub_chunks = math.ceil(per_sub_chunks / 4) * 4
    e_pad = _NSUB * per_sub_chunks * _CHUNK
    rows_per_tile = math.ceil(n / _NTILE)
    rows_per_tile = math.ceil(rows_per_tile / _CHUNK) * _CHUNK
    n_pad = rows_per_tile * _NTILE
    if e_pad > e and n_pad == n:
        n_pad += _NTILE * _CHUNK
    src = edge_index[0]
    dst = edge_index[1]
    pad = e_pad - e
    srcp = jnp.concatenate(
        [src, jnp.zeros((pad,), jnp.int32)]).reshape(_NSUB, per_sub_chunks, _CHUNK)
    pad_dst = (jnp.arange(pad, dtype=jnp.int32) % (n_pad - n)) + n
    dstp = jnp.concatenate([dst, pad_dst]).reshape(_NSUB, per_sub_chunks, _CHUNK)

    segsum = _make_segsum(n, n_pad, per_sub_chunks, hid)

    blk = 1000
    grid = n // blk
    bn2 = bn.reshape(1, hid)
    bgcn2 = bgcn.reshape(1, hid)
    bpred2 = bpred.reshape(1, hid)
    bcls2 = bcls.reshape(1, cls)

    row_spec = pl.BlockSpec((blk, hid), lambda i: (i, 0))
    w_spec = pl.BlockSpec((hid, hid), lambda i: (0, 0))
    b_spec = pl.BlockSpec((1, hid), lambda i: (0, 0))
    deg_spec = pl.BlockSpec((blk, 1), lambda i: (i, 0))
    p0_spec = pl.BlockSpec((1, blk, hid), lambda i: (0, i, 0))
    p1_spec = pl.BlockSpec((1, blk, hid), lambda i: (1, i, 0))
    cs_spec = pl.BlockSpec((1, hid), lambda i: (0, 0))

    h0, cs0 = pl.pallas_call(
        _init_body,
        grid=(grid,),
        in_specs=[row_spec, w_spec, b_spec],
        out_specs=[row_spec, cs_spec],
        out_shape=[jax.ShapeDtypeStruct((n, hid), jnp.float32),
                   jax.ShapeDtypeStruct((1, hid), jnp.float32)],
    )(node_feat, Wn, bn2)

    p = segsum(h0, srcp, dstp)
    h, hd = pl.pallas_call(
        _combine_body,
        grid=(grid,),
        in_specs=[p0_spec, p1_spec, deg_spec],
        out_specs=[row_spec, row_spec],
        out_shape=[jax.ShapeDtypeStruct((n, hid), jnp.float32),
                   jax.ShapeDtypeStruct((n, hid), jnp.float32)],
    )(p, p, degree)

    css = [cs0]
    for li in range(layers):
        p = segsum(hd, srcp, dstp)
        need_hd = li < layers - 1
        out_specs = [row_spec] + ([row_spec] if need_hd else []) + [cs_spec]
        out_shape = ([jax.ShapeDtypeStruct((n, hid), jnp.float32)]
                     + ([jax.ShapeDtypeStruct((n, hid), jnp.float32)] if need_hd else [])
                     + [jax.ShapeDtypeStruct((1, hid), jnp.float32)])
        outs = pl.pallas_call(
            _make_layer_body(need_hd),
            grid=(grid,),
            in_specs=[p0_spec, p1_spec, row_spec, deg_spec, w_spec, b_spec],
            out_specs=out_specs,
            out_shape=out_shape,
        )(p, p, h, degree, Wgcn, bgcn2)
        if need_hd:
            h, hd, cs = outs
        else:
            h, cs = outs
        css.append(cs)

    wps = [Wpred[k * hid:(k + 1) * hid] for k in range(layers + 1)]

    def _full(shape):
        return pl.BlockSpec(shape, lambda: (0,) * len(shape))

    out = pl.pallas_call(
        _make_head_body(n),
        in_specs=[_full((1, hid))] * 4 + [_full((hid, hid))] * 4
                 + [_full((1, hid)), _full((hid, cls)), _full((1, cls))],
        out_specs=_full((1, cls)),
        out_shape=jax.ShapeDtypeStruct((1, cls), jnp.float32),
    )(*css, *wps, bpred2, Wcls, bcls2)
    return out


# speedup vs baseline: 1.1823x; 1.1823x over previous
import functools
import math

import jax
import jax.numpy as jnp
from jax import lax
from jax.experimental import pallas as pl
from jax.experimental.pallas import tpu as pltpu
from jax.experimental.pallas import tpu_sc as plsc

_NSC = 2
_NTILE = 16
_NSUB = _NSC * _NTILE
_CHUNK = 128
_LEAK = 0.01


@functools.lru_cache(maxsize=None)
def _make_segsum(n_nodes, n_pad, q0, q1, hid):
    rows_per_tile = n_pad // _NTILE
    zc = rows_per_tile // _CHUNK
    k_stage = q1
    mesh = plsc.VectorSubcoreMesh(core_axis_name="c", subcore_axis_name="s")

    @functools.partial(
        pl.kernel,
        mesh=mesh,
        out_type=jax.ShapeDtypeStruct((_NSC, n_pad, hid), jnp.float32),
        scratch_types=[
            pltpu.VMEM((k_stage, _CHUNK), jnp.int32),
            pltpu.VMEM((k_stage, _CHUNK), jnp.int32),
            pltpu.VMEM((_CHUNK, hid), jnp.float32),
            pltpu.VMEM((_CHUNK, hid), jnp.float32),
            pltpu.VMEM_SHARED((n_pad, hid), jnp.float32),
            pltpu.SemaphoreType.DMA,
            pltpu.SemaphoreType.DMA,
        ],
    )
    def segsum(hd_hbm, srcp_hbm, dstp_hbm, out_hbm,
               src_v, dst_v, buf0, buf1, acc, sem0, sem1):
        c = lax.axis_index("c")
        s = lax.axis_index("s")
        w = c * _NTILE + s

        def _zrow(r, carry):
            for cc in range(hid // 16):
                buf0[r, pl.ds(cc * 16, 16)] = jnp.zeros((16,), jnp.float32)
            return carry
        lax.fori_loop(0, _CHUNK, _zrow, 0)
        base = s * rows_per_tile
        for k in range(zc):
            pltpu.sync_copy(buf0, acc.at[pl.ds(base + k * _CHUNK, _CHUNK)])
        plsc.subcore_barrier()

        def _group(g, carry):
            j0 = g * 2
            cp0 = pltpu.async_copy(hd_hbm.at[src_v.at[j0]], buf0, sem0)
            cp1 = pltpu.async_copy(hd_hbm.at[src_v.at[j0 + 1]], buf1, sem1)
            cp0.wait()
            pltpu.sync_copy(buf0, acc.at[dst_v.at[j0]], add=True)
            cp1.wait()
            pltpu.sync_copy(buf1, acc.at[dst_v.at[j0 + 1]], add=True)
            return carry

        @pl.when(c == 0)
        def _():
            for st in range(3):
                pltpu.sync_copy(srcp_hbm.at[w, pl.ds(st * k_stage, k_stage)], src_v)
                pltpu.sync_copy(dstp_hbm.at[w, pl.ds(st * k_stage, k_stage)], dst_v)
                lax.fori_loop(0, k_stage // 2, _group, 0)

        @pl.when(c == 1)
        def _():
            pltpu.sync_copy(srcp_hbm.at[w, pl.ds(0, k_stage)], src_v)
            pltpu.sync_copy(dstp_hbm.at[w, pl.ds(0, k_stage)], dst_v)
            lax.fori_loop(0, k_stage // 2, _group, 0)
        plsc.subcore_barrier()

        for k in range(zc):
            pltpu.sync_copy(acc.at[pl.ds(base + k * _CHUNK, _CHUNK)], buf0)
            pltpu.sync_copy(buf0, out_hbm.at[c, pl.ds(base + k * _CHUNK, _CHUNK)])

    return segsum


def _init_body(nf_ref, wn_ref, bn_ref, h0_ref, cs_ref):
    i = pl.program_id(0)
    h0 = jnp.dot(nf_ref[...], wn_ref[...],
                 preferred_element_type=jnp.float32) + bn_ref[...]
    h0_ref[...] = h0

    @pl.when(i == 0)
    def _():
        cs_ref[...] = jnp.zeros_like(cs_ref)
    cs_ref[...] += jnp.sum(h0, axis=0, keepdims=True)


def _combine_body(p0_ref, p1_ref, deg_ref, h1_ref, hd1_ref):
    h1 = p0_ref[0] + p1_ref[0]
    h1_ref[...] = h1
    hd1_ref[...] = h1 / deg_ref[...]


def _make_layer_body(need_hd):
    def _layer_body(p0_ref, p1_ref, h_ref, deg_ref, w_ref, b_ref, *out_refs):
        i = pl.program_id(0)
        agg = p0_ref[0] + p1_ref[0] + h_ref[...]
        hn = jnp.dot(agg, w_ref[...],
                     preferred_element_type=jnp.float32) + b_ref[...]
        hn = jnp.where(hn > 0, hn, _LEAK * hn)
        if need_hd:
            hn_ref, hdn_ref, cs_ref = out_refs
            hdn_ref[...] = hn / deg_ref[...]
        else:
            hn_ref, cs_ref = out_refs
        hn_ref[...] = hn

        @pl.when(i == 0)
        def _():
            cs_ref[...] = jnp.zeros_like(cs_ref)
        cs_ref[...] += jnp.sum(hn, axis=0, keepdims=True)
    return _layer_body


def _make_head_body(n_nodes):
    def _head_body(c0, c1, c2, c3, wp0, wp1, wp2, wp3, bp, wc, bc, out_ref):
        inv_n = 1.0 / n_nodes
        feat = (jnp.dot(c0[...] * inv_n, wp0[...], preferred_element_type=jnp.float32)
                + jnp.dot(c1[...] * inv_n, wp1[...], preferred_element_type=jnp.float32)
                + jnp.dot(c2[...] * inv_n, wp2[...], preferred_element_type=jnp.float32)
                + jnp.dot(c3[...] * inv_n, wp3[...], preferred_element_type=jnp.float32)
                + bp[...])
        logits = jnp.dot(feat, wc[...], preferred_element_type=jnp.float32) + bc[...]
        m = jnp.max(logits, axis=-1, keepdims=True)
        ex = jnp.exp(logits - m)
        out_ref[...] = ex / jnp.sum(ex, axis=-1, keepdims=True)
    return _head_body


def kernel(node_feat, edge_feat, edge_index, degree, base_data,
           Wn, bn, We, be, Wgcn, bgcn, Wpred, bpred, Wcls, bcls):
    n, d_node = node_feat.shape
    hid = Wn.shape[1]
    e = edge_index.shape[1]
    cls = Wcls.shape[1]
    layers = Wpred.shape[0] // hid - 1

    unit = math.ceil(e / (_NTILE * _CHUNK * 4))
    unit = math.ceil(unit / 8) * 8
    q0, q1 = 3 * unit, unit
    e_pad = _NTILE * (q0 + q1) * _CHUNK
    rows_per_tile = math.ceil(n / _NTILE)
    rows_per_tile = math.ceil(rows_per_tile / _CHUNK) * _CHUNK
    n_pad = rows_per_tile * _NTILE
    if e_pad > e and n_pad == n:
        n_pad += _NTILE * _CHUNK
    src = edge_index[0]
    dst = edge_index[1]
    pad = e_pad - e
    pad_dst = (jnp.arange(pad, dtype=jnp.int32) % (n_pad - n)) + n

    def _split(flat):
        ch = flat.reshape(-1, _CHUNK)
        a = ch[:_NTILE * q0].reshape(_NTILE, q0, _CHUNK)
        b = ch[_NTILE * q0:].reshape(_NTILE, q1, _CHUNK)
        b = jnp.pad(b, ((0, 0), (0, q0 - q1), (0, 0)))
        return jnp.concatenate([a, b], axis=0)

    srcp = _split(jnp.concatenate([src, jnp.zeros((pad,), jnp.int32)]))
    dstp = _split(jnp.concatenate([dst, pad_dst]))

    segsum = _make_segsum(n, n_pad, q0, q1, hid)

    blk = 1000
    grid = n // blk
    bn2 = bn.reshape(1, hid)
    bgcn2 = bgcn.reshape(1, hid)
    bpred2 = bpred.reshape(1, hid)
    bcls2 = bcls.reshape(1, cls)

    row_spec = pl.BlockSpec((blk, hid), lambda i: (i, 0))
    w_spec = pl.BlockSpec((hid, hid), lambda i: (0, 0))
    b_spec = pl.BlockSpec((1, hid), lambda i: (0, 0))
    deg_spec = pl.BlockSpec((blk, 1), lambda i: (i, 0))
    p0_spec = pl.BlockSpec((1, blk, hid), lambda i: (0, i, 0))
    p1_spec = pl.BlockSpec((1, blk, hid), lambda i: (1, i, 0))
    cs_spec = pl.BlockSpec((1, hid), lambda i: (0, 0))

    h0, cs0 = pl.pallas_call(
        _init_body,
        grid=(grid,),
        in_specs=[row_spec, w_spec, b_spec],
        out_specs=[row_spec, cs_spec],
        out_shape=[jax.ShapeDtypeStruct((n, hid), jnp.float32),
                   jax.ShapeDtypeStruct((1, hid), jnp.float32)],
    )(node_feat, Wn, bn2)

    p = segsum(h0, srcp, dstp)
    h, hd = pl.pallas_call(
        _combine_body,
        grid=(grid,),
        in_specs=[p0_spec, p1_spec, deg_spec],
        out_specs=[row_spec, row_spec],
        out_shape=[jax.ShapeDtypeStruct((n, hid), jnp.float32),
                   jax.ShapeDtypeStruct((n, hid), jnp.float32)],
    )(p, p, degree)

    css = [cs0]
    for li in range(layers):
        p = segsum(hd, srcp, dstp)
        need_hd = li < layers - 1
        out_specs = [row_spec] + ([row_spec] if need_hd else []) + [cs_spec]
        out_shape = ([jax.ShapeDtypeStruct((n, hid), jnp.float32)]
                     + ([jax.ShapeDtypeStruct((n, hid), jnp.float32)] if need_hd else [])
                     + [jax.ShapeDtypeStruct((1, hid), jnp.float32)])
        outs = pl.pallas_call(
            _make_layer_body(need_hd),
            grid=(grid,),
            in_specs=[p0_spec, p1_spec, row_spec, deg_spec, w_spec, b_spec],
            out_specs=out_specs,
            out_shape=out_shape,
        )(p, p, h, degree, Wgcn, bgcn2)
        if need_hd:
            h, hd, cs = outs
        else:
            h, cs = outs
        css.append(cs)

    wps = [Wpred[k * hid:(k + 1) * hid] for k in range(layers + 1)]

    def _full(shape):
        return pl.BlockSpec(shape, lambda: (0,) * len(shape))

    out = pl.pallas_call(
        _make_head_body(n),
        in_specs=[_full((1, hid))] * 4 + [_full((hid, hid))] * 4
                 + [_full((1, hid)), _full((hid, cls)), _full((1, cls))],
        out_specs=_full((1, cls)),
        out_shape=jax.ShapeDtypeStruct((1, cls), jnp.float32),
    )(*css, *wps, bpred2, Wcls, bcls2)
    return out
